# Initial kernel scaffold; baseline (speedup 1.0000x reference)
import jax
import jax.numpy as jnp
from jax.experimental import pallas as pl


def kernel(memory, start_embedding, W_ih, W_hh, b_ih, b_hh, W_k, W_q, v_att):
    raise NotImplementedError("write your pallas kernel here")


# speedup vs baseline: 1.4489x; 1.4489x over previous
import jax
import jax.numpy as jnp
from jax.experimental import pallas as pl
from jax.experimental.pallas import tpu as pltpu

B, S, D, H, T = 64, 2048, 512, 256, 16
NS = 4
SBLK = S // NS
BC = 4
KC = 8
_BF = jnp.bfloat16
_INTERPRET = False


def _pre_kernel(wk_ref, vt_ref, mem_ref, p8_ref, c0_ref):
    kt = jax.lax.dot_general(
        wk_ref[...].astype(_BF), mem_ref[0].astype(_BF),
        (((0,), (1,)), ((), ())), preferred_element_type=jnp.float32)
    tt = jnp.tanh(kt)
    p8_ref[0] = jnp.round((1.0 - tt * tt) * 127.0).astype(jnp.int8)
    c0_ref[0] = jnp.sum(tt * vt_ref[...], axis=0, keepdims=True)


def _precompute(memory, W_k, vT):
    return pl.pallas_call(
        _pre_kernel,
        grid=(B,),
        in_specs=[pl.BlockSpec((D, H), lambda i: (0, 0)),
                  pl.BlockSpec((H, 1), lambda i: (0, 0)),
                  pl.BlockSpec((1, S, D), lambda i: (i, 0, 0))],
        out_specs=[pl.BlockSpec((1, H, S), lambda i: (i, 0, 0)),
                   pl.BlockSpec((1, 1, S), lambda i: (i, 0, 0))],
        out_shape=[jax.ShapeDtypeStruct((B, H, S), jnp.int8),
                   jax.ShapeDtypeStruct((B, 1, S), jnp.float32)],
        interpret=_INTERPRET,
    )(W_k, vT, memory)


def _decode_kernel(p8_ref, c0_ref, mem_ref, start_ref, wih_ref, whh_ref,
                   bias_ref, wq_ref, wk_ref, v2_ref, vt_ref,
                   preds_ref, logps_ref,
                   h_ref, c_ref, q_ref, x_ref, w3_ref, ss_ref, mask_ref,
                   cand_ref, mrows_ref, p8v_ref, cs_ref, sem_p, sem_s, sem_g):
    t = pl.program_id(0)

    @pl.when(t == 0)
    def _init():
        cpp = pltpu.make_async_copy(p8_ref, p8v_ref, sem_p)
        cpp.start()
        h_ref[...] = jnp.zeros((B, H), jnp.float32)
        c_ref[...] = jnp.zeros((B, H), jnp.float32)
        x_ref[...] = start_ref[...]
        for jb in range(NS):
            mask_ref[jb] = jnp.zeros((B, SBLK), jnp.float32)
        cpp.wait()

    x = x_ref[...]
    h = h_ref[...]
    c = c_ref[...]
    gates = (jax.lax.dot_general(x.astype(_BF), wih_ref[...].astype(_BF),
                                 (((1,), (1,)), ((), ())),
                                 preferred_element_type=jnp.float32)
             + jax.lax.dot_general(h.astype(_BF), whh_ref[...].astype(_BF),
                                   (((1,), (1,)), ((), ())),
                                   preferred_element_type=jnp.float32)
             + bias_ref[...])
    i_g = gates[:, 0:H]
    f_g = gates[:, H:2 * H]
    g_g = gates[:, 2 * H:3 * H]
    o_g = gates[:, 3 * H:4 * H]
    c_new = jax.nn.sigmoid(f_g) * c + jax.nn.sigmoid(i_g) * jnp.tanh(g_g)
    h_new = jax.nn.sigmoid(o_g) * jnp.tanh(c_new)
    h_ref[...] = h_new
    c_ref[...] = c_new
    q = jax.lax.dot_general(h_new.astype(_BF), wq_ref[...].astype(_BF),
                            (((1,), (0,)), ((), ())),
                            preferred_element_type=jnp.float32)
    q_ref[...] = q
    w3_ref[...] = ((q * v2_ref[...]) * (1.0 / 127.0))[:, :, None]

    w3 = w3_ref[...]
    for jb in range(NS):
        lo = jb * SBLK
        for bc in range(B // BC):
            pb = p8v_ref[bc * BC:(bc + 1) * BC, :, lo:lo + SBLK]
            wb = w3[bc * BC:(bc + 1) * BC]
            sv = jnp.sum(pb.astype(jnp.float32) * wb, axis=1)
            sv = sv + c0_ref[bc * BC:(bc + 1) * BC, lo:lo + SBLK]
            mk = mask_ref[jb, bc * BC:(bc + 1) * BC, :]
            ss_ref[jb, bc * BC:(bc + 1) * BC, :] = jnp.where(
                mk > 0.5, jnp.float32(-1e9), sv)

    mt = jnp.full((B, 1), -3e38, jnp.float32)
    for jb in range(NS):
        mt = jnp.maximum(mt, jnp.max(ss_ref[jb], axis=1, keepdims=True))
    se = jnp.zeros((B, 1), jnp.float32)
    for jb in range(NS):
        se = se + jnp.sum(jnp.exp(ss_ref[jb] - mt), axis=1, keepdims=True)
    lse = mt + jnp.log(se)

    iota = jax.lax.broadcasted_iota(jnp.int32, (B, SBLK), 1)
    for i in range(KC):
        gm = jnp.full((B, 1), -3e38, jnp.float32)
        for jb in range(NS):
            gm = jnp.maximum(gm, jnp.max(ss_ref[jb], axis=1, keepdims=True))
        idx = jnp.full((B, 1), 2 ** 30, jnp.int32)
        for jb in range(NS):
            idx = jnp.minimum(idx, jnp.min(
                jnp.where(ss_ref[jb] >= gm, iota + jb * SBLK,
                          jnp.int32(2 ** 30)), axis=1, keepdims=True))
        cand_ref[i] = idx
        for jb in range(NS):
            ss_ref[jb] = jnp.where(iota + jb * SBLK == idx,
                                   jnp.float32(-3e38), ss_ref[jb])

    cps = pltpu.make_async_copy(cand_ref, cs_ref, sem_s)
    cps.start()
    cps.wait()
    copies = []
    for i in range(KC):
        for b in range(B):
            copies.append(pltpu.make_async_copy(
                mem_ref.at[b, cs_ref[i, b, 0]],
                mrows_ref.at[i * B + b], sem_g))
    for c_ in copies:
        c_.start()
    for c_ in copies:
        c_.wait()

    mrows = mrows_ref[...]
    krows = jax.lax.dot_general(mrows.astype(_BF), wk_ref[...].astype(_BF),
                                (((1,), (0,)), ((), ())),
                                preferred_element_type=jnp.float32)
    qrep = jnp.broadcast_to(q[None], (KC, B, H)).reshape(KC * B, H)
    tc = jnp.tanh(qrep + krows)
    sc = jax.lax.dot_general(tc.astype(_BF), vt_ref[...].astype(_BF),
                             (((1,), (0,)), ((), ())),
                             preferred_element_type=jnp.float32)
    sc3 = sc.reshape(KC, B, 1)
    cand = cand_ref[...]
    mx = jnp.max(sc3, axis=0)
    hit = sc3 >= mx[None]
    pred = jnp.min(jnp.where(hit, cand, jnp.int32(2 ** 30)), axis=0)
    win = jnp.logical_and(hit, cand == pred[None])
    xsel = jnp.sum(jnp.where(win, 1.0, 0.0)
                   * mrows.reshape(KC, B, D), axis=0)
    x_ref[...] = xsel

    preds_ref[0] = pred
    logps_ref[0] = mx - lse
    for jb in range(NS):
        io2 = jax.lax.broadcasted_iota(jnp.int32, (B, SBLK), 1) + jb * SBLK
        mask_ref[jb] = jnp.where(io2 == pred, 1.0, mask_ref[jb])


def _decode(p8, c0, memory, start_embedding, W_ih, W_hh, bias, W_q, W_k,
            v2, vT):
    return pl.pallas_call(
        _decode_kernel,
        grid=(T,),
        in_specs=[
            pl.BlockSpec(memory_space=pl.ANY),
            pl.BlockSpec((B, S), lambda t: (0, 0)),
            pl.BlockSpec(memory_space=pl.ANY),
            pl.BlockSpec((B, D), lambda t: (0, 0)),
            pl.BlockSpec((4 * H, D), lambda t: (0, 0)),
            pl.BlockSpec((4 * H, H), lambda t: (0, 0)),
            pl.BlockSpec((1, 4 * H), lambda t: (0, 0)),
            pl.BlockSpec((H, H), lambda t: (0, 0)),
            pl.BlockSpec((D, H), lambda t: (0, 0)),
            pl.BlockSpec((1, H), lambda t: (0, 0)),
            pl.BlockSpec((H, 1), lambda t: (0, 0)),
        ],
        out_specs=[pl.BlockSpec((1, B, 1), lambda t: (t, 0, 0)),
                   pl.BlockSpec((1, B, 1), lambda t: (t, 0, 0))],
        out_shape=[jax.ShapeDtypeStruct((T, B, 1), jnp.int32),
                   jax.ShapeDtypeStruct((T, B, 1), jnp.float32)],
        scratch_shapes=[
            pltpu.VMEM((B, H), jnp.float32),
            pltpu.VMEM((B, H), jnp.float32),
            pltpu.VMEM((B, H), jnp.float32),
            pltpu.VMEM((B, D), jnp.float32),
            pltpu.VMEM((B, H, 1), jnp.float32),
            pltpu.VMEM((NS, B, SBLK), jnp.float32),
            pltpu.VMEM((NS, B, SBLK), jnp.float32),
            pltpu.VMEM((KC, B, 1), jnp.int32),
            pltpu.VMEM((KC * B, D), jnp.float32),
            pltpu.VMEM((B, H, S), jnp.int8),
            pltpu.SMEM((KC, B, 1), jnp.int32),
            pltpu.SemaphoreType.DMA,
            pltpu.SemaphoreType.DMA,
            pltpu.SemaphoreType.DMA,
        ],
        compiler_params=pltpu.CompilerParams(
            dimension_semantics=("arbitrary",),
            vmem_limit_bytes=63 * 1024 * 1024,
        ),
        interpret=_INTERPRET,
    )(p8, c0, memory, start_embedding, W_ih, W_hh, bias, W_q, W_k, v2, vT)


def kernel(memory, start_embedding, W_ih, W_hh, b_ih, b_hh, W_k, W_q, v_att):
    vT = v_att.reshape(H, 1)
    v2 = v_att.reshape(1, H)
    p8, c03 = _precompute(memory, W_k, vT)
    c0 = c03.reshape(B, S)
    bias = (b_ih + b_hh).reshape(1, 4 * H)
    preds3, logps3 = _decode(p8, c0, memory, start_embedding, W_ih, W_hh,
                             bias, W_q, W_k, v2, vT)
    predictions = preds3.reshape(T, B).T
    log_probs = logps3.reshape(T, B).T
    return predictions, log_probs


# speedup vs baseline: 1.6065x; 1.1088x over previous
import jax
import jax.numpy as jnp
from jax.experimental import pallas as pl
from jax.experimental.pallas import tpu as pltpu

B, S, D, H, T = 64, 2048, 512, 256, 16
NS = 4
SBLK = S // NS
BC = 4
KC = 4
_BF = jnp.bfloat16
_INTERPRET = False


def _pre_kernel(wk_ref, vt_ref, mem_ref, p8_ref, c0_ref):
    kt = jax.lax.dot_general(
        wk_ref[...].astype(_BF), mem_ref[0].astype(_BF),
        (((0,), (1,)), ((), ())), preferred_element_type=jnp.float32)
    tt = jnp.tanh(kt)
    p8_ref[0] = jnp.round((1.0 - tt * tt) * 127.0).astype(jnp.int8)
    c0_ref[0] = jnp.sum(tt * vt_ref[...], axis=0, keepdims=True)


def _precompute(memory, W_k, vT):
    return pl.pallas_call(
        _pre_kernel,
        grid=(B,),
        in_specs=[pl.BlockSpec((D, H), lambda i: (0, 0)),
                  pl.BlockSpec((H, 1), lambda i: (0, 0)),
                  pl.BlockSpec((1, S, D), lambda i: (i, 0, 0))],
        out_specs=[pl.BlockSpec((1, H, S), lambda i: (i, 0, 0)),
                   pl.BlockSpec((1, 1, S), lambda i: (i, 0, 0))],
        out_shape=[jax.ShapeDtypeStruct((B, H, S), jnp.int8),
                   jax.ShapeDtypeStruct((B, 1, S), jnp.float32)],
        interpret=_INTERPRET,
    )(W_k, vT, memory)


def _decode_kernel(p8_ref, c0_ref, mem_ref, start_ref, wih_ref, whh_ref,
                   bias_ref, wq_ref, wk_ref, v2_ref, vt_ref,
                   preds_ref, logps_ref,
                   h_ref, c_ref, q_ref, x_ref, w3_ref, ss_ref, mask_ref,
                   cand_ref, mrows_ref, p8v_ref, cs_ref, sem_p, sem_s, sem_g):
    t = pl.program_id(0)

    @pl.when(t == 0)
    def _init():
        cpp = pltpu.make_async_copy(p8_ref, p8v_ref, sem_p)
        cpp.start()
        h_ref[...] = jnp.zeros((B, H), jnp.float32)
        c_ref[...] = jnp.zeros((B, H), jnp.float32)
        x_ref[...] = start_ref[...]
        for jb in range(NS):
            mask_ref[jb] = jnp.zeros((B, SBLK), jnp.float32)
        cpp.wait()

    x = x_ref[...]
    h = h_ref[...]
    c = c_ref[...]
    gates = (jax.lax.dot_general(x.astype(_BF), wih_ref[...].astype(_BF),
                                 (((1,), (1,)), ((), ())),
                                 preferred_element_type=jnp.float32)
             + jax.lax.dot_general(h.astype(_BF), whh_ref[...].astype(_BF),
                                   (((1,), (1,)), ((), ())),
                                   preferred_element_type=jnp.float32)
             + bias_ref[...])
    i_g = gates[:, 0:H]
    f_g = gates[:, H:2 * H]
    g_g = gates[:, 2 * H:3 * H]
    o_g = gates[:, 3 * H:4 * H]
    c_new = jax.nn.sigmoid(f_g) * c + jax.nn.sigmoid(i_g) * jnp.tanh(g_g)
    h_new = jax.nn.sigmoid(o_g) * jnp.tanh(c_new)
    h_ref[...] = h_new
    c_ref[...] = c_new
    q = jax.lax.dot_general(h_new.astype(_BF), wq_ref[...].astype(_BF),
                            (((1,), (0,)), ((), ())),
                            preferred_element_type=jnp.float32)
    q_ref[...] = q
    w3_ref[...] = ((q * v2_ref[...]) * (1.0 / 127.0))[:, :, None]

    w3 = w3_ref[...]
    for jb in range(NS):
        lo = jb * SBLK
        for bc in range(B // BC):
            pb = p8v_ref[bc * BC:(bc + 1) * BC, :, lo:lo + SBLK]
            wb = w3[bc * BC:(bc + 1) * BC]
            sv = jnp.sum(pb.astype(jnp.float32) * wb, axis=1)
            sv = sv + c0_ref[bc * BC:(bc + 1) * BC, lo:lo + SBLK]
            mk = mask_ref[jb, bc * BC:(bc + 1) * BC, :]
            ss_ref[jb, bc * BC:(bc + 1) * BC, :] = jnp.where(
                mk > 0.5, jnp.float32(-1e9), sv)

    iota = jax.lax.broadcasted_iota(jnp.int32, (B, SBLK), 1)
    gms = []
    for i in range(KC):
        gm = jnp.full((B, 1), -3e38, jnp.float32)
        for jb in range(NS):
            gm = jnp.maximum(gm, jnp.max(ss_ref[jb], axis=1, keepdims=True))
        gms.append(gm)
        idx = jnp.full((B, 1), 2 ** 30, jnp.int32)
        for jb in range(NS):
            idx = jnp.minimum(idx, jnp.min(
                jnp.where(ss_ref[jb] >= gm, iota + jb * SBLK,
                          jnp.int32(2 ** 30)), axis=1, keepdims=True))
        cand_ref[i] = idx
        for jb in range(NS):
            ss_ref[jb] = jnp.where(iota + jb * SBLK == idx,
                                   jnp.float32(-3e38), ss_ref[jb])

    cps = pltpu.make_async_copy(cand_ref, cs_ref, sem_s)
    cps.start()
    cps.wait()
    copies = []
    for i in range(KC):
        for b in range(B):
            copies.append(pltpu.make_async_copy(
                mem_ref.at[b, cs_ref[i, b, 0]],
                mrows_ref.at[i * B + b], sem_g))
    for c_ in copies:
        c_.start()

    mt = gms[0]
    se = jnp.zeros((B, 1), jnp.float32)
    for jb in range(NS):
        se = se + jnp.sum(jnp.exp(ss_ref[jb] - mt), axis=1, keepdims=True)
    for gm in gms:
        se = se + jnp.exp(gm - mt)
    lse = mt + jnp.log(se)

    for c_ in copies:
        c_.wait()

    mrows = mrows_ref[...]
    krows = jax.lax.dot_general(mrows.astype(_BF), wk_ref[...].astype(_BF),
                                (((1,), (0,)), ((), ())),
                                preferred_element_type=jnp.float32)
    qrep = jnp.broadcast_to(q[None], (KC, B, H)).reshape(KC * B, H)
    tc = jnp.tanh(qrep + krows)
    sc = jax.lax.dot_general(tc.astype(_BF), vt_ref[...].astype(_BF),
                             (((1,), (0,)), ((), ())),
                             preferred_element_type=jnp.float32)
    sc3 = sc.reshape(KC, B, 1)
    cand = cand_ref[...]
    mx = jnp.max(sc3, axis=0)
    hit = sc3 >= mx[None]
    pred = jnp.min(jnp.where(hit, cand, jnp.int32(2 ** 30)), axis=0)
    win = jnp.logical_and(hit, cand == pred[None])
    xsel = jnp.sum(jnp.where(win, 1.0, 0.0)
                   * mrows.reshape(KC, B, D), axis=0)
    x_ref[...] = xsel

    preds_ref[0] = pred
    logps_ref[0] = mx - lse
    for jb in range(NS):
        io2 = jax.lax.broadcasted_iota(jnp.int32, (B, SBLK), 1) + jb * SBLK
        mask_ref[jb] = jnp.where(io2 == pred, 1.0, mask_ref[jb])


def _decode(p8, c0, memory, start_embedding, W_ih, W_hh, bias, W_q, W_k,
            v2, vT):
    return pl.pallas_call(
        _decode_kernel,
        grid=(T,),
        in_specs=[
            pl.BlockSpec(memory_space=pl.ANY),
            pl.BlockSpec((B, S), lambda t: (0, 0)),
            pl.BlockSpec(memory_space=pl.ANY),
            pl.BlockSpec((B, D), lambda t: (0, 0)),
            pl.BlockSpec((4 * H, D), lambda t: (0, 0)),
            pl.BlockSpec((4 * H, H), lambda t: (0, 0)),
            pl.BlockSpec((1, 4 * H), lambda t: (0, 0)),
            pl.BlockSpec((H, H), lambda t: (0, 0)),
            pl.BlockSpec((D, H), lambda t: (0, 0)),
            pl.BlockSpec((1, H), lambda t: (0, 0)),
            pl.BlockSpec((H, 1), lambda t: (0, 0)),
        ],
        out_specs=[pl.BlockSpec((1, B, 1), lambda t: (t, 0, 0)),
                   pl.BlockSpec((1, B, 1), lambda t: (t, 0, 0))],
        out_shape=[jax.ShapeDtypeStruct((T, B, 1), jnp.int32),
                   jax.ShapeDtypeStruct((T, B, 1), jnp.float32)],
        scratch_shapes=[
            pltpu.VMEM((B, H), jnp.float32),
            pltpu.VMEM((B, H), jnp.float32),
            pltpu.VMEM((B, H), jnp.float32),
            pltpu.VMEM((B, D), jnp.float32),
            pltpu.VMEM((B, H, 1), jnp.float32),
            pltpu.VMEM((NS, B, SBLK), jnp.float32),
            pltpu.VMEM((NS, B, SBLK), jnp.float32),
            pltpu.VMEM((KC, B, 1), jnp.int32),
            pltpu.VMEM((KC * B, D), jnp.float32),
            pltpu.VMEM((B, H, S), jnp.int8),
            pltpu.SMEM((KC, B, 1), jnp.int32),
            pltpu.SemaphoreType.DMA,
            pltpu.SemaphoreType.DMA,
            pltpu.SemaphoreType.DMA,
        ],
        compiler_params=pltpu.CompilerParams(
            dimension_semantics=("arbitrary",),
            vmem_limit_bytes=63 * 1024 * 1024,
        ),
        interpret=_INTERPRET,
    )(p8, c0, memory, start_embedding, W_ih, W_hh, bias, W_q, W_k, v2, vT)


def kernel(memory, start_embedding, W_ih, W_hh, b_ih, b_hh, W_k, W_q, v_att):
    vT = v_att.reshape(H, 1)
    v2 = v_att.reshape(1, H)
    p8, c03 = _precompute(memory, W_k, vT)
    c0 = c03.reshape(B, S)
    bias = (b_ih + b_hh).reshape(1, 4 * H)
    preds3, logps3 = _decode(p8, c0, memory, start_embedding, W_ih, W_hh,
                             bias, W_q, W_k, v2, vT)
    predictions = preds3.reshape(T, B).T
    log_probs = logps3.reshape(T, B).T
    return predictions, log_probs


# speedup vs baseline: 1.6159x; 1.0059x over previous
import jax
import jax.numpy as jnp
from jax.experimental import pallas as pl
from jax.experimental.pallas import tpu as pltpu

B, S, D, H, T = 64, 2048, 512, 256, 16
NS = 4
SBLK = S // NS
BC = 4
KC = 4
_BF = jnp.bfloat16
_INTERPRET = False


def _pre_kernel(wk_ref, vt_ref, mem_ref, p8_ref, c0_ref):
    kt = jax.lax.dot_general(
        wk_ref[...].astype(_BF), mem_ref[0].astype(_BF),
        (((0,), (1,)), ((), ())), preferred_element_type=jnp.float32)
    tt = jnp.tanh(kt)
    p8_ref[0] = jnp.round((1.0 - tt * tt) * 127.0).astype(jnp.int8)
    c0_ref[0] = jnp.sum(tt * vt_ref[...], axis=0, keepdims=True)


def _precompute(memory, W_k, vT):
    return pl.pallas_call(
        _pre_kernel,
        grid=(B,),
        in_specs=[pl.BlockSpec((D, H), lambda i: (0, 0)),
                  pl.BlockSpec((H, 1), lambda i: (0, 0)),
                  pl.BlockSpec((1, S, D), lambda i: (i, 0, 0))],
        out_specs=[pl.BlockSpec((1, H, S), lambda i: (i, 0, 0)),
                   pl.BlockSpec((1, 1, S), lambda i: (i, 0, 0))],
        out_shape=[jax.ShapeDtypeStruct((B, H, S), jnp.int8),
                   jax.ShapeDtypeStruct((B, 1, S), jnp.float32)],
        interpret=_INTERPRET,
    )(W_k, vT, memory)


def _decode_kernel(p8_ref, c0_ref, mem_ref, start_ref, wih_ref, whh_ref,
                   bias_ref, wq_ref, wk_ref, v2_ref, vt_ref,
                   preds_ref, logps_ref,
                   h_ref, c_ref, q_ref, x_ref, w3_ref, ss_ref, mask_ref,
                   cand_ref, mrows_ref, p8v_ref, cs_ref, sem_p, sem_s, sem_g):
    t = pl.program_id(0)

    @pl.when(t == 0)
    def _init():
        cpp = pltpu.make_async_copy(p8_ref, p8v_ref, sem_p)
        cpp.start()
        h_ref[...] = jnp.zeros((B, H), jnp.float32)
        c_ref[...] = jnp.zeros((B, H), jnp.float32)
        x_ref[...] = start_ref[...]
        for jb in range(NS):
            mask_ref[jb] = jnp.zeros((B, SBLK), jnp.float32)
        cpp.wait()

    x = x_ref[...]
    h = h_ref[...]
    c = c_ref[...]
    gates = (jax.lax.dot_general(x.astype(_BF), wih_ref[...].astype(_BF),
                                 (((1,), (1,)), ((), ())),
                                 preferred_element_type=jnp.float32)
             + jax.lax.dot_general(h.astype(_BF), whh_ref[...].astype(_BF),
                                   (((1,), (1,)), ((), ())),
                                   preferred_element_type=jnp.float32)
             + bias_ref[...])
    i_g = gates[:, 0:H]
    f_g = gates[:, H:2 * H]
    g_g = gates[:, 2 * H:3 * H]
    o_g = gates[:, 3 * H:4 * H]
    c_new = jax.nn.sigmoid(f_g) * c + jax.nn.sigmoid(i_g) * jnp.tanh(g_g)
    h_new = jax.nn.sigmoid(o_g) * jnp.tanh(c_new)
    h_ref[...] = h_new
    c_ref[...] = c_new
    q = jax.lax.dot_general(h_new.astype(_BF), wq_ref[...].astype(_BF),
                            (((1,), (0,)), ((), ())),
                            preferred_element_type=jnp.float32)
    q_ref[...] = q
    w3_ref[...] = ((q * v2_ref[...]) * (1.0 / 127.0))[:, :, None]

    w3 = w3_ref[...]
    for jb in range(NS):
        lo = jb * SBLK
        for bc in range(B // BC):
            pb = p8v_ref[bc * BC:(bc + 1) * BC, :, lo:lo + SBLK]
            wb = w3[bc * BC:(bc + 1) * BC]
            sv = jnp.sum(pb.astype(jnp.float32) * wb, axis=1)
            sv = sv + c0_ref[bc * BC:(bc + 1) * BC, lo:lo + SBLK]
            mk = mask_ref[jb, bc * BC:(bc + 1) * BC, :]
            ss_ref[jb, bc * BC:(bc + 1) * BC, :] = jnp.where(
                mk > 0.5, jnp.float32(-1e9), sv)

    iota = jax.lax.broadcasted_iota(jnp.int32, (B, SBLK), 1)
    gms = []
    for i in range(KC):
        gm = jnp.full((B, 1), -3e38, jnp.float32)
        for jb in range(NS):
            gm = jnp.maximum(gm, jnp.max(ss_ref[jb], axis=1, keepdims=True))
        gms.append(gm)
        idx = jnp.full((B, 1), 2 ** 30, jnp.int32)
        for jb in range(NS):
            idx = jnp.minimum(idx, jnp.min(
                jnp.where(ss_ref[jb] >= gm, iota + jb * SBLK,
                          jnp.int32(2 ** 30)), axis=1, keepdims=True))
        cand_ref[i] = idx
        for jb in range(NS):
            ss_ref[jb] = jnp.where(iota + jb * SBLK == idx,
                                   jnp.float32(-3e38), ss_ref[jb])

    cps = pltpu.make_async_copy(cand_ref, cs_ref, sem_s)
    cps.start()
    cps.wait()
    copies = []
    for i in range(KC):
        for b in range(B):
            copies.append(pltpu.make_async_copy(
                mem_ref.at[b, cs_ref[i, b, 0]],
                mrows_ref.at[i * B + b], sem_g))
    for c_ in copies:
        c_.start()

    mt = gms[0]
    se = jnp.zeros((B, 1), jnp.float32)
    for jb in range(NS):
        se = se + jnp.sum(jnp.exp(ss_ref[jb] - mt), axis=1, keepdims=True)
    for gm in gms:
        se = se + jnp.exp(gm - mt)
    lse = mt + jnp.log(se)

    for c_ in copies:
        c_.wait()

    mrows = mrows_ref[...]
    krows = jax.lax.dot_general(mrows.astype(_BF), wk_ref[...].astype(_BF),
                                (((1,), (0,)), ((), ())),
                                preferred_element_type=jnp.float32)
    qrep = jnp.broadcast_to(q[None], (KC, B, H)).reshape(KC * B, H)
    tc = jnp.tanh(qrep + krows)
    sc = jax.lax.dot_general(tc.astype(_BF), vt_ref[...].astype(_BF),
                             (((1,), (0,)), ((), ())),
                             preferred_element_type=jnp.float32)
    sc3 = sc.reshape(KC, B, 1)
    cand = cand_ref[...]
    mx = jnp.max(sc3, axis=0)
    hit = sc3 >= mx[None]
    pred = jnp.min(jnp.where(hit, cand, jnp.int32(2 ** 30)), axis=0)
    win = jnp.logical_and(hit, cand == pred[None])
    xsel = jnp.sum(jnp.where(win, 1.0, 0.0)
                   * mrows.reshape(KC, B, D), axis=0)
    x_ref[...] = xsel

    preds_ref[0] = pred
    logps_ref[0] = mx - lse
    for jb in range(NS):
        io2 = jax.lax.broadcasted_iota(jnp.int32, (B, SBLK), 1) + jb * SBLK
        mask_ref[jb] = jnp.where(io2 == pred, 1.0, mask_ref[jb])


def _decode(p8, c0, memory, start_embedding, W_ih, W_hh, bias, W_q, W_k,
            v2, vT):
    return pl.pallas_call(
        _decode_kernel,
        grid=(T,),
        in_specs=[
            pl.BlockSpec(memory_space=pl.ANY),
            pl.BlockSpec((B, S), lambda t: (0, 0)),
            pl.BlockSpec(memory_space=pl.ANY),
            pl.BlockSpec((B, D), lambda t: (0, 0)),
            pl.BlockSpec((4 * H, D), lambda t: (0, 0)),
            pl.BlockSpec((4 * H, H), lambda t: (0, 0)),
            pl.BlockSpec((1, 4 * H), lambda t: (0, 0)),
            pl.BlockSpec((H, H), lambda t: (0, 0)),
            pl.BlockSpec((D, H), lambda t: (0, 0)),
            pl.BlockSpec((1, H), lambda t: (0, 0)),
            pl.BlockSpec((H, 1), lambda t: (0, 0)),
        ],
        out_specs=[pl.BlockSpec((1, B, 1), lambda t: (t, 0, 0)),
                   pl.BlockSpec((1, B, 1), lambda t: (t, 0, 0))],
        out_shape=[jax.ShapeDtypeStruct((T, B, 1), jnp.int32),
                   jax.ShapeDtypeStruct((T, B, 1), jnp.float32)],
        scratch_shapes=[
            pltpu.VMEM((B, H), jnp.float32),
            pltpu.VMEM((B, H), jnp.float32),
            pltpu.VMEM((B, H), jnp.float32),
            pltpu.VMEM((B, D), jnp.float32),
            pltpu.VMEM((B, H, 1), jnp.float32),
            pltpu.VMEM((NS, B, SBLK), jnp.float32),
            pltpu.VMEM((NS, B, SBLK), jnp.float32),
            pltpu.VMEM((KC, B, 1), jnp.int32),
            pltpu.VMEM((KC * B, D), jnp.float32),
            pltpu.VMEM((B, H, S), jnp.int8),
            pltpu.SMEM((KC, B, 1), jnp.int32),
            pltpu.SemaphoreType.DMA,
            pltpu.SemaphoreType.DMA,
            pltpu.SemaphoreType.DMA,
        ],
        compiler_params=pltpu.CompilerParams(
            dimension_semantics=("arbitrary",),
            vmem_limit_bytes=63 * 1024 * 1024,
            disable_bounds_checks=True,
        ),
        interpret=_INTERPRET,
    )(p8, c0, memory, start_embedding, W_ih, W_hh, bias, W_q, W_k, v2, vT)


def kernel(memory, start_embedding, W_ih, W_hh, b_ih, b_hh, W_k, W_q, v_att):
    vT = v_att.reshape(H, 1)
    v2 = v_att.reshape(1, H)
    p8, c03 = _precompute(memory, W_k, vT)
    c0 = c03.reshape(B, S)
    bias = (b_ih + b_hh).reshape(1, 4 * H)
    preds3, logps3 = _decode(p8, c0, memory, start_embedding, W_ih, W_hh,
                             bias, W_q, W_k, v2, vT)
    predictions = preds3.reshape(T, B).T
    log_probs = logps3.reshape(T, B).T
    return predictions, log_probs


# speedup vs baseline: 1.9507x; 1.2072x over previous
import jax
import jax.numpy as jnp
from jax.experimental import pallas as pl
from jax.experimental.pallas import tpu as pltpu

B, S, D, H, T = 64, 2048, 512, 256, 16
NS = 4
SBLK = S // NS
BC = 4
KC = 4
_BF = jnp.bfloat16
_INTERPRET = False


def _pre_kernel(wk_ref, vt_ref, mem_ref, p8_ref, c0_ref):
    kt = jax.lax.dot_general(
        wk_ref[...].astype(_BF), mem_ref[0].astype(_BF),
        (((0,), (1,)), ((), ())), preferred_element_type=jnp.float32)
    tt = jnp.tanh(kt)
    p8_ref[0] = jnp.round((1.0 - tt * tt) * 127.0).astype(jnp.int8)
    c0_ref[0] = jnp.sum(tt * vt_ref[...], axis=0, keepdims=True)


def _precompute(memory, W_k, vT):
    return pl.pallas_call(
        _pre_kernel,
        grid=(B,),
        in_specs=[pl.BlockSpec((D, H), lambda i: (0, 0)),
                  pl.BlockSpec((H, 1), lambda i: (0, 0)),
                  pl.BlockSpec((1, S, D), lambda i: (i, 0, 0))],
        out_specs=[pl.BlockSpec((1, H, S), lambda i: (i, 0, 0)),
                   pl.BlockSpec((1, 1, S), lambda i: (i, 0, 0))],
        out_shape=[jax.ShapeDtypeStruct((B, H, S), jnp.int8),
                   jax.ShapeDtypeStruct((B, 1, S), jnp.float32)],
        interpret=_INTERPRET,
    )(W_k, vT, memory)


def _decode_kernel(p8_ref, c0_ref, mem_ref, start_ref, wih_ref, whh_ref,
                   bias_ref, wq_ref, wk_ref, v2_ref, vt_ref,
                   preds_ref, logps_ref,
                   h_ref, c_ref, q_ref, x_ref, w3_ref, ss_ref, mask_ref,
                   cand_ref, mrows_ref, p8v_ref, cs_ref, sem_p, sem_s, sem_g):
    t = pl.program_id(0)

    @pl.when(t == 0)
    def _init():
        cpp = pltpu.make_async_copy(p8_ref, p8v_ref, sem_p)
        cpp.start()
        h_ref[...] = jnp.zeros((B, H), jnp.float32)
        c_ref[...] = jnp.zeros((B, H), jnp.float32)
        x_ref[...] = start_ref[...]
        for jb in range(NS):
            mask_ref[jb] = jnp.zeros((B, SBLK), jnp.float32)
        cpp.wait()

    x = x_ref[...]
    h = h_ref[...]
    c = c_ref[...]
    gates = (jax.lax.dot_general(x.astype(_BF), wih_ref[...].astype(_BF),
                                 (((1,), (1,)), ((), ())),
                                 preferred_element_type=jnp.float32)
             + jax.lax.dot_general(h.astype(_BF), whh_ref[...].astype(_BF),
                                   (((1,), (1,)), ((), ())),
                                   preferred_element_type=jnp.float32)
             + bias_ref[...])
    i_g = gates[:, 0:H]
    f_g = gates[:, H:2 * H]
    g_g = gates[:, 2 * H:3 * H]
    o_g = gates[:, 3 * H:4 * H]
    c_new = jax.nn.sigmoid(f_g) * c + jax.nn.sigmoid(i_g) * jnp.tanh(g_g)
    h_new = jax.nn.sigmoid(o_g) * jnp.tanh(c_new)
    h_ref[...] = h_new
    c_ref[...] = c_new
    q = jax.lax.dot_general(h_new.astype(_BF), wq_ref[...].astype(_BF),
                            (((1,), (0,)), ((), ())),
                            preferred_element_type=jnp.float32)
    q_ref[...] = q
    w3_ref[...] = jnp.round((q * v2_ref[...]) * 32768.0
                            ).astype(jnp.int32)[:, :, None]

    w3 = w3_ref[...]
    for jb in range(NS):
        lo = jb * SBLK
        for bc in range(B // BC):
            pb = p8v_ref[bc * BC:(bc + 1) * BC, :, lo:lo + SBLK]
            wb = w3[bc * BC:(bc + 1) * BC]
            acc = jnp.sum(pb.astype(jnp.int32) * wb, axis=1)
            sv = (acc.astype(jnp.float32) * (1.0 / (127.0 * 32768.0))
                  + c0_ref[bc * BC:(bc + 1) * BC, lo:lo + SBLK])
            mk = mask_ref[jb, bc * BC:(bc + 1) * BC, :]
            ss_ref[jb, bc * BC:(bc + 1) * BC, :] = jnp.where(
                mk > 0.5, jnp.float32(-1e9), sv)

    iota = jax.lax.broadcasted_iota(jnp.int32, (B, SBLK), 1)
    gms = []
    for i in range(KC):
        gm = jnp.full((B, 1), -3e38, jnp.float32)
        for jb in range(NS):
            gm = jnp.maximum(gm, jnp.max(ss_ref[jb], axis=1, keepdims=True))
        gms.append(gm)
        idx = jnp.full((B, 1), 2 ** 30, jnp.int32)
        for jb in range(NS):
            idx = jnp.minimum(idx, jnp.min(
                jnp.where(ss_ref[jb] >= gm, iota + jb * SBLK,
                          jnp.int32(2 ** 30)), axis=1, keepdims=True))
        cand_ref[i] = idx
        for jb in range(NS):
            ss_ref[jb] = jnp.where(iota + jb * SBLK == idx,
                                   jnp.float32(-3e38), ss_ref[jb])

    cps = pltpu.make_async_copy(cand_ref, cs_ref, sem_s)
    cps.start()
    cps.wait()
    copies = []
    for i in range(KC):
        for b in range(B):
            copies.append(pltpu.make_async_copy(
                mem_ref.at[b, cs_ref[i, b, 0]],
                mrows_ref.at[i * B + b], sem_g))
    for c_ in copies:
        c_.start()

    mt = gms[0]
    se = jnp.zeros((B, 1), jnp.float32)
    for jb in range(NS):
        se = se + jnp.sum(jnp.exp(ss_ref[jb] - mt), axis=1, keepdims=True)
    for gm in gms:
        se = se + jnp.exp(gm - mt)
    lse = mt + jnp.log(se)

    for c_ in copies:
        c_.wait()

    mrows = mrows_ref[...]
    krows = jax.lax.dot_general(mrows.astype(_BF), wk_ref[...].astype(_BF),
                                (((1,), (0,)), ((), ())),
                                preferred_element_type=jnp.float32)
    qrep = jnp.broadcast_to(q[None], (KC, B, H)).reshape(KC * B, H)
    tc = jnp.tanh(qrep + krows)
    sc = jax.lax.dot_general(tc.astype(_BF), vt_ref[...].astype(_BF),
                             (((1,), (0,)), ((), ())),
                             preferred_element_type=jnp.float32)
    sc3 = sc.reshape(KC, B, 1)
    cand = cand_ref[...]
    mx = jnp.max(sc3, axis=0)
    hit = sc3 >= mx[None]
    pred = jnp.min(jnp.where(hit, cand, jnp.int32(2 ** 30)), axis=0)
    win = jnp.logical_and(hit, cand == pred[None])
    xsel = jnp.sum(jnp.where(win, 1.0, 0.0)
                   * mrows.reshape(KC, B, D), axis=0)
    x_ref[...] = xsel

    preds_ref[0] = pred
    logps_ref[0] = mx - lse
    for jb in range(NS):
        io2 = jax.lax.broadcasted_iota(jnp.int32, (B, SBLK), 1) + jb * SBLK
        mask_ref[jb] = jnp.where(io2 == pred, 1.0, mask_ref[jb])


def _decode(p8, c0, memory, start_embedding, W_ih, W_hh, bias, W_q, W_k,
            v2, vT):
    return pl.pallas_call(
        _decode_kernel,
        grid=(T,),
        in_specs=[
            pl.BlockSpec(memory_space=pl.ANY),
            pl.BlockSpec((B, S), lambda t: (0, 0)),
            pl.BlockSpec(memory_space=pl.ANY),
            pl.BlockSpec((B, D), lambda t: (0, 0)),
            pl.BlockSpec((4 * H, D), lambda t: (0, 0)),
            pl.BlockSpec((4 * H, H), lambda t: (0, 0)),
            pl.BlockSpec((1, 4 * H), lambda t: (0, 0)),
            pl.BlockSpec((H, H), lambda t: (0, 0)),
            pl.BlockSpec((D, H), lambda t: (0, 0)),
            pl.BlockSpec((1, H), lambda t: (0, 0)),
            pl.BlockSpec((H, 1), lambda t: (0, 0)),
        ],
        out_specs=[pl.BlockSpec((1, B, 1), lambda t: (t, 0, 0)),
                   pl.BlockSpec((1, B, 1), lambda t: (t, 0, 0))],
        out_shape=[jax.ShapeDtypeStruct((T, B, 1), jnp.int32),
                   jax.ShapeDtypeStruct((T, B, 1), jnp.float32)],
        scratch_shapes=[
            pltpu.VMEM((B, H), jnp.float32),
            pltpu.VMEM((B, H), jnp.float32),
            pltpu.VMEM((B, H), jnp.float32),
            pltpu.VMEM((B, D), jnp.float32),
            pltpu.VMEM((B, H, 1), jnp.int32),
            pltpu.VMEM((NS, B, SBLK), jnp.float32),
            pltpu.VMEM((NS, B, SBLK), jnp.float32),
            pltpu.VMEM((KC, B, 1), jnp.int32),
            pltpu.VMEM((KC * B, D), jnp.float32),
            pltpu.VMEM((B, H, S), jnp.int8),
            pltpu.SMEM((KC, B, 1), jnp.int32),
            pltpu.SemaphoreType.DMA,
            pltpu.SemaphoreType.DMA,
            pltpu.SemaphoreType.DMA,
        ],
        compiler_params=pltpu.CompilerParams(
            dimension_semantics=("arbitrary",),
            vmem_limit_bytes=63 * 1024 * 1024,
            disable_bounds_checks=True,
        ),
        interpret=_INTERPRET,
    )(p8, c0, memory, start_embedding, W_ih, W_hh, bias, W_q, W_k, v2, vT)


def kernel(memory, start_embedding, W_ih, W_hh, b_ih, b_hh, W_k, W_q, v_att):
    vT = v_att.reshape(H, 1)
    v2 = v_att.reshape(1, H)
    p8, c03 = _precompute(memory, W_k, vT)
    c0 = c03.reshape(B, S)
    bias = (b_ih + b_hh).reshape(1, 4 * H)
    preds3, logps3 = _decode(p8, c0, memory, start_embedding, W_ih, W_hh,
                             bias, W_q, W_k, v2, vT)
    predictions = preds3.reshape(T, B).T
    log_probs = logps3.reshape(T, B).T
    return predictions, log_probs
